# baseline (speedup 1.0000x reference)
import functools

import jax
import jax.numpy as jnp
from jax import lax
from jax.experimental import pallas as pl
from jax.experimental.pallas import tpu as pltpu
from jax.experimental.pallas import tpu_sc as plsc

_VOCAB = 100000
_DIM = 64
_B = 16384
_L = 50
_N = _B * _L
_NW = 32
_PER_W = _N // _NW
_B_PER_W = _B // _NW
_NB = 16
_CHUNK = _NB * _L
_NCHUNK = _PER_W // _CHUNK
_NBUF = 2
_NGROUP = _NCHUNK // _NBUF


def _make_gather():
    mesh = plsc.VectorSubcoreMesh(core_axis_name="c", subcore_axis_name="s")

    @functools.partial(
        pl.kernel,
        mesh=mesh,
        out_type=jax.ShapeDtypeStruct((_B, _L, _DIM), jnp.float32),
        scratch_types=[
            pltpu.VMEM((_CHUNK,), jnp.int32),
            pltpu.VMEM((_CHUNK,), jnp.int32),
            pltpu.VMEM((_CHUNK, _DIM), jnp.float32),
            pltpu.VMEM((_CHUNK, _DIM), jnp.float32),
            pltpu.SemaphoreType.DMA,
            pltpu.SemaphoreType.DMA,
            pltpu.SemaphoreType.DMA,
            pltpu.SemaphoreType.DMA,
            pltpu.SemaphoreType.DMA,
            pltpu.SemaphoreType.DMA,
        ],
        compiler_params=pltpu.CompilerParams(use_tc_tiling_on_sc=False),
    )
    def gather_kernel(table_hbm, idx_hbm, out_hbm, idx_v0, idx_v1,
                      rows_v0, rows_v1,
                      isem0, isem1, gsem0, gsem1, osem0, osem1):
        idx_vs = (idx_v0, idx_v1)
        rows_vs = (rows_v0, rows_v1)
        isems = (isem0, isem1)
        gsems = (gsem0, gsem1)
        osems = (osem0, osem1)
        wid = lax.axis_index("s") * 2 + lax.axis_index("c")
        base = wid * _PER_W
        bbase = wid * _B_PER_W

        def idx_chunk(i):
            return idx_hbm.at[pl.ds(base + i * _CHUNK, _CHUNK)]

        def store_chunk(i, b, start):
            b0 = bbase + i * _NB
            for j in range(_NB):
                cp = pltpu.make_async_copy(
                    rows_vs[b].at[pl.ds(j * _L, _L)], out_hbm.at[b0 + j], osems[b])
                if start:
                    cp.start()
                else:
                    cp.wait()

        for b in range(_NBUF):
            pltpu.async_copy(idx_chunk(b), idx_vs[b], isems[b])

        def group(g, carry):
            for b in range(_NBUF):
                i = g * _NBUF + b
                pltpu.make_async_copy(idx_chunk(i), idx_vs[b], isems[b]).wait()

                @pl.when(g >= 1)
                def _():
                    store_chunk(i, b, start=False)

                pltpu.async_copy(table_hbm.at[idx_vs[b]], rows_vs[b], gsems[b]).wait()

                @pl.when(g < _NGROUP - 1)
                def _():
                    pltpu.async_copy(idx_chunk(i + _NBUF), idx_vs[b], isems[b])

                store_chunk(i, b, start=True)
            return carry

        lax.fori_loop(0, _NGROUP, group, 0)

        for b in range(_NBUF):
            store_chunk(b, b, start=False)

    return gather_kernel


_gather = _make_gather()


@jax.jit
def kernel(input, embd_weight):
    idx_flat = input.reshape(_N).astype(jnp.int32)
    return _gather(embd_weight, idx_flat)
